# baseline (speedup 1.0000x reference)
import functools

import jax
import jax.numpy as jnp
from jax import lax
from jax.experimental import pallas as pl
from jax.experimental.pallas import tpu as pltpu
from jax.experimental.pallas import tpu_sc as plsc

NC = 2
NS = 16
NW = NC * NS

BATCH = 4
SEQ = 2048
D_MODEL = 768
N_TOK = BATCH * SEQ
B_PER_W = N_TOK // NW
CHUNK = 16
N_CHUNKS = B_PER_W // CHUNK
NBUF = 8
W_PER_B = SEQ // B_PER_W


def _embed_kernel(table_hbm, tok_hbm, out_hbm,
                  idx_v, buf0, buf1, buf2, buf3, buf4, buf5, buf6, buf7,
                  g0, g1, g2, g3, g4, g5, g6, g7,
                  s0, s1, s2, s3, s4, s5, s6, s7):
    wid = lax.axis_index("s") * NC + lax.axis_index("c")
    brow = wid // W_PER_B
    col = (wid % W_PER_B) * B_PER_W

    bufs = (buf0, buf1, buf2, buf3, buf4, buf5, buf6, buf7)
    gsems = (g0, g1, g2, g3, g4, g5, g6, g7)
    ssems = (s0, s1, s2, s3, s4, s5, s6, s7)

    pltpu.sync_copy(tok_hbm.at[brow, pl.ds(col, B_PER_W)], idx_v)

    def start_gather(chunk, slot):
        return pltpu.async_copy(
            table_hbm.at[idx_v.at[pl.ds(chunk * CHUNK, CHUNK)]],
            bufs[slot], gsems[slot])

    def start_scatter(chunk, slot):
        return pltpu.async_copy(
            bufs[slot],
            out_hbm.at[brow, pl.ds(col + chunk * CHUNK, CHUNK)],
            ssems[slot])

    gathers = {}
    scatters = {}
    for b in range(NBUF):
        gathers[b] = start_gather(b, b)

    for j in range(N_CHUNKS):
        slot = j % NBUF
        p = j - 1
        if p >= 0 and p + NBUF < N_CHUNKS:
            scatters[p].wait()
            gathers[p + NBUF] = start_gather(p + NBUF, p % NBUF)
        gathers[j].wait()
        scatters[j] = start_scatter(j, slot)

    for p in range(max(0, N_CHUNKS - NBUF), N_CHUNKS):
        scatters[p].wait()


@jax.jit
def kernel(tokens, W_E):
    mesh = plsc.VectorSubcoreMesh(
        core_axis_name="c", subcore_axis_name="s",
        num_cores=NC, num_subcores=NS)
    run = functools.partial(
        pl.kernel,
        out_type=jax.ShapeDtypeStruct((BATCH, SEQ, D_MODEL), jnp.float32),
        mesh=mesh,
        scratch_types=[
            pltpu.VMEM((B_PER_W,), jnp.int32),
            pltpu.VMEM((CHUNK, D_MODEL), jnp.float32),
            pltpu.VMEM((CHUNK, D_MODEL), jnp.float32),
            pltpu.VMEM((CHUNK, D_MODEL), jnp.float32),
            pltpu.VMEM((CHUNK, D_MODEL), jnp.float32),
            pltpu.VMEM((CHUNK, D_MODEL), jnp.float32),
            pltpu.VMEM((CHUNK, D_MODEL), jnp.float32),
            pltpu.VMEM((CHUNK, D_MODEL), jnp.float32),
            pltpu.VMEM((CHUNK, D_MODEL), jnp.float32),
            pltpu.SemaphoreType.DMA,
            pltpu.SemaphoreType.DMA,
            pltpu.SemaphoreType.DMA,
            pltpu.SemaphoreType.DMA,
            pltpu.SemaphoreType.DMA,
            pltpu.SemaphoreType.DMA,
            pltpu.SemaphoreType.DMA,
            pltpu.SemaphoreType.DMA,
            pltpu.SemaphoreType.DMA,
            pltpu.SemaphoreType.DMA,
            pltpu.SemaphoreType.DMA,
            pltpu.SemaphoreType.DMA,
            pltpu.SemaphoreType.DMA,
            pltpu.SemaphoreType.DMA,
            pltpu.SemaphoreType.DMA,
            pltpu.SemaphoreType.DMA,
        ],
    )(_embed_kernel)
    return run(W_E, tokens)


# speedup vs baseline: 1.0291x; 1.0291x over previous
import functools

import jax
import jax.numpy as jnp
from jax import lax
from jax.experimental import pallas as pl
from jax.experimental.pallas import tpu as pltpu
from jax.experimental.pallas import tpu_sc as plsc

NC = 2
NS = 16
NW = NC * NS

BATCH = 4
SEQ = 2048
D_MODEL = 768
N_TOK = BATCH * SEQ
B_PER_W = N_TOK // NW
CHUNK = 32
N_CHUNKS = B_PER_W // CHUNK
NBUF = 5
W_PER_B = SEQ // B_PER_W


def _embed_kernel(table_hbm, tok_hbm, out_hbm,
                  idx_v, buf0, buf1, buf2, buf3, buf4,
                  g0, g1, g2, g3, g4, s0, s1, s2, s3, s4):
    wid = lax.axis_index("s") * NC + lax.axis_index("c")
    brow = wid // W_PER_B
    col = (wid % W_PER_B) * B_PER_W

    bufs = (buf0, buf1, buf2, buf3, buf4)
    gsems = (g0, g1, g2, g3, g4)
    ssems = (s0, s1, s2, s3, s4)

    head = 4 * CHUNK
    pltpu.sync_copy(tok_hbm.at[brow, pl.ds(col, head)],
                    idx_v.at[pl.ds(0, head)])

    def start_gather(chunk, slot):
        return pltpu.async_copy(
            table_hbm.at[idx_v.at[pl.ds(chunk * CHUNK, CHUNK)]],
            bufs[slot], gsems[slot])

    def start_scatter(chunk, slot):
        return pltpu.async_copy(
            bufs[slot],
            out_hbm.at[brow, pl.ds(col + chunk * CHUNK, CHUNK)],
            ssems[slot])

    gathers = {}
    scatters = {}
    for b in range(4):
        gathers[b] = start_gather(b, b)
    pltpu.sync_copy(tok_hbm.at[brow, pl.ds(col + head, B_PER_W - head)],
                    idx_v.at[pl.ds(head, B_PER_W - head)])
    for b in range(4, NBUF):
        gathers[b] = start_gather(b, b)

    for j in range(N_CHUNKS):
        slot = j % NBUF
        p = j - 1
        if p >= 0 and p + NBUF < N_CHUNKS:
            scatters[p].wait()
            gathers[p + NBUF] = start_gather(p + NBUF, p % NBUF)
        gathers[j].wait()
        scatters[j] = start_scatter(j, slot)

    for p in range(max(0, N_CHUNKS - NBUF), N_CHUNKS):
        scatters[p].wait()


@jax.jit
def kernel(tokens, W_E):
    mesh = plsc.VectorSubcoreMesh(
        core_axis_name="c", subcore_axis_name="s",
        num_cores=NC, num_subcores=NS)
    run = functools.partial(
        pl.kernel,
        out_type=jax.ShapeDtypeStruct((BATCH, SEQ, D_MODEL), jnp.float32),
        mesh=mesh,
        scratch_types=[
            pltpu.VMEM((B_PER_W,), jnp.int32),
            pltpu.VMEM((CHUNK, D_MODEL), jnp.float32),
            pltpu.VMEM((CHUNK, D_MODEL), jnp.float32),
            pltpu.VMEM((CHUNK, D_MODEL), jnp.float32),
            pltpu.VMEM((CHUNK, D_MODEL), jnp.float32),
            pltpu.VMEM((CHUNK, D_MODEL), jnp.float32),
            pltpu.SemaphoreType.DMA,
            pltpu.SemaphoreType.DMA,
            pltpu.SemaphoreType.DMA,
            pltpu.SemaphoreType.DMA,
            pltpu.SemaphoreType.DMA,
            pltpu.SemaphoreType.DMA,
            pltpu.SemaphoreType.DMA,
            pltpu.SemaphoreType.DMA,
            pltpu.SemaphoreType.DMA,
            pltpu.SemaphoreType.DMA,
        ],
    )(_embed_kernel)
    return run(W_E, tokens)
